# Initial kernel scaffold; baseline (speedup 1.0000x reference)
import functools

import jax
import jax.numpy as jnp
from jax import lax
from jax.experimental import pallas as pl
from jax.experimental.pallas import tpu as pltpu, tpu_sc as plsc

_B = 1024
_U = 50
_P = 20
_D = 32
_N = _B * _U

_NC = 2
_NS = 16
_NW = _NC * _NS
_ROWS_W = _N // _NW
_C = 64
_NCHUNK = _ROWS_W // _C
_G = _C * _P

_mesh = plsc.VectorSubcoreMesh(core_axis_name="c", subcore_axis_name="s")


@functools.partial(
    pl.kernel,
    out_type=jax.ShapeDtypeStruct((_N, _D), jnp.float32),
    mesh=_mesh,
    scratch_types=[
        pltpu.VMEM((_G,), jnp.int32),
        pltpu.VMEM((_G, _D), jnp.float32),
        pltpu.VMEM((_C, _D), jnp.float32),
        pltpu.SemaphoreType.DMA,
    ],
)
def _gather_sum(idx_hbm, table_hbm, out_hbm, idx_v, rows_v, out_v, sem):
    wid = lax.axis_index("s") * _NC + lax.axis_index("c")
    w_base = wid * _ROWS_W

    def chunk_body(k, _):
        base = w_base + k * _C
        pltpu.sync_copy(idx_hbm.at[pl.ds(base * _P, _G)], idx_v)
        pltpu.async_copy(table_hbm.at[idx_v], rows_v, sem).wait()

        def acc_body(j, _):
            r = j * _P
            a0 = rows_v[r, pl.ds(0, 16)]
            a1 = rows_v[r, pl.ds(16, 16)]
            for g in range(1, _P):
                a0 = a0 + rows_v[r + g, pl.ds(0, 16)]
                a1 = a1 + rows_v[r + g, pl.ds(16, 16)]
            out_v[j, pl.ds(0, 16)] = a0
            out_v[j, pl.ds(16, 16)] = a1
            return 0

        lax.fori_loop(0, _C, acc_body, 0)
        pltpu.sync_copy(out_v, out_hbm.at[pl.ds(base, _C)])
        return 0

    lax.fori_loop(0, _NCHUNK, chunk_body, 0)


def kernel(prev_cmd, num_cmd, ctype_emb_weight):
    del num_cmd
    table = ctype_emb_weight.at[0].set(0.0)
    idx = prev_cmd.reshape(-1).astype(jnp.int32)
    out = _gather_sum(idx, table)
    return out.reshape(_B, _U, _D)


# speedup vs baseline: 18.5203x; 18.5203x over previous
import functools

import jax
import jax.numpy as jnp
from jax import lax
from jax.experimental import pallas as pl
from jax.experimental.pallas import tpu as pltpu, tpu_sc as plsc

_B = 1024
_U = 50
_P = 20
_D = 32
_N = _B * _U

_NC = 2
_NS = 16
_NW = _NC * _NS
_ROWS_W = _N // _NW
_C = 64
_NCHUNK = _ROWS_W // _C
_G = _C * _P

_mesh = plsc.VectorSubcoreMesh(core_axis_name="c", subcore_axis_name="s")


@functools.partial(
    pl.kernel,
    out_type=jax.ShapeDtypeStruct((_N, _D), jnp.float32),
    mesh=_mesh,
    compiler_params=pltpu.CompilerParams(use_tc_tiling_on_sc=False),
    scratch_types=[
        pltpu.VMEM((_G,), jnp.int32),
        pltpu.VMEM((_G, _D), jnp.float32),
        pltpu.VMEM((_C, _D), jnp.float32),
        pltpu.SemaphoreType.DMA,
    ],
)
def _gather_sum(idx_hbm, table_hbm, out_hbm, idx_v, rows_v, out_v, sem):
    wid = lax.axis_index("s") * _NC + lax.axis_index("c")
    w_base = wid * _ROWS_W

    def chunk_body(k, _):
        base = w_base + k * _C
        pltpu.sync_copy(idx_hbm.at[pl.ds(base * _P, _G)], idx_v)
        pltpu.async_copy(table_hbm.at[idx_v], rows_v, sem).wait()

        def acc_body(j, _):
            r = j * _P
            a0 = rows_v[r, pl.ds(0, 16)]
            a1 = rows_v[r, pl.ds(16, 16)]
            for g in range(1, _P):
                a0 = a0 + rows_v[r + g, pl.ds(0, 16)]
                a1 = a1 + rows_v[r + g, pl.ds(16, 16)]
            out_v[j, pl.ds(0, 16)] = a0
            out_v[j, pl.ds(16, 16)] = a1
            return 0

        lax.fori_loop(0, _C, acc_body, 0)
        pltpu.sync_copy(out_v, out_hbm.at[pl.ds(base, _C)])
        return 0

    lax.fori_loop(0, _NCHUNK, chunk_body, 0)


def kernel(prev_cmd, num_cmd, ctype_emb_weight):
    del num_cmd
    table = ctype_emb_weight.at[0].set(0.0)
    idx = prev_cmd.reshape(-1).astype(jnp.int32)
    out = _gather_sum(idx, table)
    return out.reshape(_B, _U, _D)


# speedup vs baseline: 23.7709x; 1.2835x over previous
import functools

import jax
import jax.numpy as jnp
from jax import lax
from jax.experimental import pallas as pl
from jax.experimental.pallas import tpu as pltpu, tpu_sc as plsc

_B = 1024
_U = 50
_P = 20
_D = 32
_N = _B * _U

_NC = 2
_NS = 16
_NW = _NC * _NS
_ROWS_W = _N // _NW
_C = 200
_NCHUNK = _ROWS_W // _C
_NBLK = _NW * _NCHUNK

_mesh = plsc.VectorSubcoreMesh(core_axis_name="c", subcore_axis_name="s")


@functools.partial(
    pl.kernel,
    out_type=jax.ShapeDtypeStruct((_N, _D), jnp.float32),
    mesh=_mesh,
    compiler_params=pltpu.CompilerParams(use_tc_tiling_on_sc=False),
    scratch_types=[
        pltpu.VMEM((_P, _C), jnp.int32),
        pltpu.VMEM((_C, _D), jnp.float32),
        pltpu.SemaphoreType.DMA,
    ],
)
def _gather_sum(idx_hbm, table_hbm, out_hbm, idx_v, acc_v, sem):
    wid = lax.axis_index("s") * _NC + lax.axis_index("c")

    def chunk_body(k, _):
        blk = wid * _NCHUNK + k
        pltpu.sync_copy(idx_hbm.at[blk], idx_v)

        zero = jnp.zeros((16,), jnp.float32)

        def zero_body(j, _):
            acc_v[j, pl.ds(0, 16)] = zero
            acc_v[j, pl.ds(16, 16)] = zero
            return 0

        lax.fori_loop(0, _C, zero_body, 0)

        descs = [
            pltpu.async_copy(table_hbm.at[idx_v.at[g]], acc_v, sem, add=True)
            for g in range(_P)
        ]
        for d in descs:
            d.wait()
        pltpu.sync_copy(acc_v, out_hbm.at[pl.ds(blk * _C, _C)])
        return 0

    lax.fori_loop(0, _NCHUNK, chunk_body, 0)


def kernel(prev_cmd, num_cmd, ctype_emb_weight):
    del num_cmd
    table = ctype_emb_weight.at[0].set(0.0)
    idx = (
        prev_cmd.astype(jnp.int32)
        .reshape(_NW, _NCHUNK, _C, _P)
        .transpose(0, 1, 3, 2)
        .reshape(_NBLK, _P, _C)
    )
    out = _gather_sum(idx, table)
    return out.reshape(_B, _U, _D)


# speedup vs baseline: 24.1935x; 1.0178x over previous
import functools

import jax
import jax.numpy as jnp
from jax import lax
from jax.experimental import pallas as pl
from jax.experimental.pallas import tpu as pltpu, tpu_sc as plsc

_B = 1024
_U = 50
_P = 20
_D = 32
_N = _B * _U

_NC = 2
_NS = 16
_NW = _NC * _NS
_ROWS_W = _N // _NW
_C = 400
_NCHUNK = _ROWS_W // _C
_CP = _C * _P
_NSLICE = _C // 16

_mesh = plsc.VectorSubcoreMesh(core_axis_name="c", subcore_axis_name="s")


@functools.partial(
    pl.kernel,
    out_type=jax.ShapeDtypeStruct((_N, _D), jnp.float32),
    mesh=_mesh,
    compiler_params=pltpu.CompilerParams(use_tc_tiling_on_sc=False,
                                         needs_layout_passes=False),
    scratch_types=[
        pltpu.VMEM((_ROWS_W * _P,), jnp.int32),
        pltpu.VMEM((_P, _C), jnp.int32),
        pltpu.VMEM((_P, _C), jnp.int32),
        pltpu.VMEM((_C, _D), jnp.float32),
        pltpu.VMEM((_C, _D), jnp.float32),
        pltpu.SemaphoreType.DMA,
        pltpu.SemaphoreType.DMA,
        pltpu.SemaphoreType.DMA,
        pltpu.SemaphoreType.DMA,
    ],
)
def _gather_sum(idx_hbm, table_hbm, out_hbm, idx_raw, it0, it1, a0, a1,
                sg0, sg1, so0, so1):
    wid = lax.axis_index("s") * _NC + lax.axis_index("c")
    w_row = wid * _ROWS_W
    its = (it0, it1)
    accs = (a0, a1)
    sgs = (sg0, sg1)
    sos = (so0, so1)

    pltpu.sync_copy(idx_hbm.at[pl.ds(w_row * _P, _ROWS_W * _P)], idx_raw)

    lane20 = jnp.arange(16, dtype=jnp.int32) * _P
    zero16 = jnp.zeros((16,), jnp.float32)

    def prep(k, b):
        it = its[b]

        def g_body(g, _):
            def s_body(s, _):
                base = k * _CP + s * (16 * _P) + g
                v = plsc.load_gather(idx_raw, [lane20 + base])
                it[g, pl.ds(s * 16, 16)] = v
                return 0

            lax.fori_loop(0, _NSLICE, s_body, 0)
            return 0

        lax.fori_loop(0, _P, g_body, 0)

        a = accs[b]

        def z_body(j, _):
            a[j, pl.ds(0, 16)] = zero16
            a[j, pl.ds(16, 16)] = zero16
            return 0

        lax.fori_loop(0, _C, z_body, 0)

    def fire(b):
        return [
            pltpu.async_copy(table_hbm.at[its[b].at[g]], accs[b], sgs[b],
                             add=True)
            for g in range(_P)
        ]

    descs = [None, None]
    out_descs = [None, None]

    prep(0, 0)
    descs[0] = fire(0)
    for k in range(_NCHUNK):
        b = k & 1
        nb = 1 - b
        if k + 1 < _NCHUNK:
            if k + 1 >= 2:
                out_descs[nb].wait()
            prep(k + 1, nb)
            descs[nb] = fire(nb)
        for d in descs[b]:
            d.wait()
        out_descs[b] = pltpu.async_copy(
            accs[b], out_hbm.at[pl.ds(w_row + k * _C, _C)], sos[b])
    out_descs[(_NCHUNK - 1) & 1].wait()
    out_descs[(_NCHUNK - 2) & 1].wait()


def kernel(prev_cmd, num_cmd, ctype_emb_weight):
    del num_cmd
    table = ctype_emb_weight.at[0].set(0.0)
    idx = prev_cmd.astype(jnp.int32).reshape(-1)
    out = _gather_sum(idx, table)
    return out.reshape(_B, _U, _D)


# speedup vs baseline: 25.2457x; 1.0435x over previous
import functools

import jax
import jax.numpy as jnp
from jax import lax
from jax.experimental import pallas as pl
from jax.experimental.pallas import tpu as pltpu, tpu_sc as plsc

_B = 1024
_U = 50
_P = 20
_D = 32
_N = _B * _U

_NC = 2
_NS = 16
_NW = _NC * _NS
_ROWS_W = _N // _NW
_BW = _B // _NW
_C = 400
_CB = _C // _U
_NCHUNK = _ROWS_W // _C
_CP = _C * _P
_NSLICE = _C // 16

_mesh = plsc.VectorSubcoreMesh(core_axis_name="c", subcore_axis_name="s")


@functools.partial(
    pl.kernel,
    out_type=jax.ShapeDtypeStruct((_B, _U, _D), jnp.float32),
    mesh=_mesh,
    compiler_params=pltpu.CompilerParams(use_tc_tiling_on_sc=False,
                                         needs_layout_passes=False),
    scratch_types=[
        pltpu.VMEM((_BW, _U, _P), jnp.int32),
        pltpu.VMEM((_P, _C), jnp.int32),
        pltpu.VMEM((_P, _C), jnp.int32),
        pltpu.VMEM((_C, _D), jnp.float32),
        pltpu.VMEM((_C, _D), jnp.float32),
        pltpu.SemaphoreType.DMA,
        pltpu.SemaphoreType.DMA,
        pltpu.SemaphoreType.DMA,
        pltpu.SemaphoreType.DMA,
    ],
)
def _gather_sum(idx_hbm, table_hbm, out_hbm, idx_raw, it0, it1, a0, a1,
                sg0, sg1, so0, so1):
    wid = lax.axis_index("s") * _NC + lax.axis_index("c")
    its = (it0, it1)
    accs = (a0, a1)
    sgs = (sg0, sg1)
    sos = (so0, so1)

    pltpu.sync_copy(idx_hbm.at[pl.ds(wid * _BW, _BW)], idx_raw)

    lane16 = jnp.arange(16, dtype=jnp.int32)
    zero16 = jnp.zeros((16,), jnp.float32)

    def prep(k, b):
        it = its[b]

        def s_body(s, _):
            r = k * _C + s * 16 + lane16
            bq = r // _U
            u = r % _U

            def g_body(g, _):
                v = plsc.load_gather(
                    idx_raw, [bq, u, jnp.full((16,), g, jnp.int32)])
                it[g, pl.ds(s * 16, 16)] = v
                return 0

            lax.fori_loop(0, _P, g_body, 0)
            return 0

        lax.fori_loop(0, _NSLICE, s_body, 0)

        a = accs[b]

        def z_body(j, _):
            a[j, pl.ds(0, 16)] = zero16
            a[j, pl.ds(16, 16)] = zero16
            return 0

        lax.fori_loop(0, _C, z_body, 0)

    def fire(b):
        return [
            pltpu.async_copy(table_hbm.at[its[b].at[g]], accs[b], sgs[b],
                             add=True)
            for g in range(_P)
        ]

    def fire_out(k, b):
        b0 = wid * _BW + k * _CB
        return [
            pltpu.async_copy(accs[b].at[pl.ds(q * _U, _U)],
                             out_hbm.at[b0 + q], sos[b])
            for q in range(_CB)
        ]

    descs = [None, None]
    out_descs = [None, None]

    prep(0, 0)
    descs[0] = fire(0)
    for k in range(_NCHUNK):
        b = k & 1
        nb = 1 - b
        if k + 1 < _NCHUNK:
            if k + 1 >= 2:
                for d in out_descs[nb]:
                    d.wait()
            prep(k + 1, nb)
            descs[nb] = fire(nb)
        for d in descs[b]:
            d.wait()
        out_descs[b] = fire_out(k, b)
    for b in ((_NCHUNK - 1) & 1, (_NCHUNK - 2) & 1):
        for d in out_descs[b]:
            d.wait()


def kernel(prev_cmd, num_cmd, ctype_emb_weight):
    del num_cmd
    table = ctype_emb_weight.at[0].set(0.0)
    return _gather_sum(prev_cmd.astype(jnp.int32), table)


# speedup vs baseline: 31.9232x; 1.2645x over previous
import functools

import jax
import jax.numpy as jnp
from jax import lax
from jax.experimental import pallas as pl
from jax.experimental.pallas import tpu as pltpu, tpu_sc as plsc

_B = 1024
_U = 50
_P = 20
_D = 32
_N = _B * _U

_NC = 2
_NS = 16
_NW = _NC * _NS
_ROWS_W = _N // _NW
_BW = _B // _NW
_C = 400
_CB = _C // _U
_NCHUNK = _ROWS_W // _C
_CP = _C * _P
_NSLICE = _C // 16

_mesh = plsc.VectorSubcoreMesh(core_axis_name="c", subcore_axis_name="s")


@functools.partial(
    pl.kernel,
    out_type=jax.ShapeDtypeStruct((_B, _U, _D), jnp.float32),
    mesh=_mesh,
    compiler_params=pltpu.CompilerParams(use_tc_tiling_on_sc=False,
                                         needs_layout_passes=False),
    scratch_types=[
        pltpu.VMEM((_P, _U, _BW), jnp.int32),
        pltpu.VMEM((_P, _C), jnp.int32),
        pltpu.VMEM((_P, _C), jnp.int32),
        pltpu.VMEM((_C, _D), jnp.float32),
        pltpu.VMEM((_C, _D), jnp.float32),
        pltpu.SemaphoreType.DMA,
        pltpu.SemaphoreType.DMA,
        pltpu.SemaphoreType.DMA,
        pltpu.SemaphoreType.DMA,
    ],
)
def _gather_sum(idx_hbm, table_hbm, out_hbm, idx_raw, it0, it1, a0, a1,
                sg0, sg1, so0, so1):
    wid = lax.axis_index("s") * _NC + lax.axis_index("c")
    its = (it0, it1)
    accs = (a0, a1)
    sgs = (sg0, sg1)
    sos = (so0, so1)

    pltpu.sync_copy(idx_hbm.at[:, :, pl.ds(wid * _BW, _BW)], idx_raw)

    lane16 = jnp.arange(16, dtype=jnp.int32)
    zero16 = jnp.zeros((16,), jnp.float32)

    def prep(k, b):
        it = its[b]

        def s_body(s, _):
            r = k * _C + s * 16 + lane16
            bq = r // _U
            u = r % _U

            def g_body(g, _):
                v = plsc.load_gather(
                    idx_raw, [jnp.full((16,), g, jnp.int32), u, bq])
                it[g, pl.ds(s * 16, 16)] = v
                return 0

            lax.fori_loop(0, _P, g_body, 0)
            return 0

        lax.fori_loop(0, _NSLICE, s_body, 0)

        a = accs[b]

        def z_body(j, _):
            a[j, pl.ds(0, 16)] = zero16
            a[j, pl.ds(16, 16)] = zero16
            return 0

        lax.fori_loop(0, _C, z_body, 0)

    def fire(b):
        return [
            pltpu.async_copy(table_hbm.at[its[b].at[g]], accs[b], sgs[b],
                             add=True)
            for g in range(_P)
        ]

    def fire_out(k, b):
        b0 = wid * _BW + k * _CB
        return [
            pltpu.async_copy(accs[b].at[pl.ds(q * _U, _U)],
                             out_hbm.at[b0 + q], sos[b])
            for q in range(_CB)
        ]

    descs = [None, None]
    out_descs = [None, None]

    prep(0, 0)
    descs[0] = fire(0)
    for k in range(_NCHUNK):
        b = k & 1
        nb = 1 - b
        if k + 1 < _NCHUNK:
            if k + 1 >= 2:
                for d in out_descs[nb]:
                    d.wait()
            prep(k + 1, nb)
            descs[nb] = fire(nb)
        for d in descs[b]:
            d.wait()
        out_descs[b] = fire_out(k, b)
    for b in ((_NCHUNK - 1) & 1, (_NCHUNK - 2) & 1):
        for d in out_descs[b]:
            d.wait()


def kernel(prev_cmd, num_cmd, ctype_emb_weight):
    del num_cmd
    table = ctype_emb_weight.at[0].set(0.0)
    idx = prev_cmd.astype(jnp.int32).transpose(2, 1, 0)
    return _gather_sum(idx, table)


# speedup vs baseline: 33.9230x; 1.0626x over previous
import functools

import jax
import jax.numpy as jnp
from jax import lax
from jax.experimental import pallas as pl
from jax.experimental.pallas import tpu as pltpu, tpu_sc as plsc

_B = 1024
_U = 50
_P = 20
_D = 32
_N = _B * _U

_NC = 2
_NS = 16
_NW = _NC * _NS
_ROWS_W = _N // _NW
_BW = _B // _NW
_C = 400
_CB = _C // _U
_NCHUNK = _ROWS_W // _C
_TB = _B // 128
_UNITS_W = _P * _TB // _NW

_mesh = plsc.VectorSubcoreMesh(core_axis_name="c", subcore_axis_name="s")


@functools.partial(
    pl.kernel,
    out_type=jax.ShapeDtypeStruct((_N * _P,), jnp.int32),
    mesh=_mesh,
    compiler_params=pltpu.CompilerParams(use_tc_tiling_on_sc=True,
                                         needs_layout_passes=False),
    scratch_types=[
        pltpu.VMEM((56, 128), jnp.int32),
        pltpu.VMEM((56, 128), jnp.int32),
        pltpu.VMEM((50 * 128,), jnp.int32),
        pltpu.VMEM((50 * 128,), jnp.int32),
        pltpu.SemaphoreType.DMA,
        pltpu.SemaphoreType.DMA,
        pltpu.SemaphoreType.DMA,
        pltpu.SemaphoreType.DMA,
    ],
)
def _detile(idx_hbm, out_hbm, i0, i1, o0, o1, si0, si1, so0, so1):
    wid = lax.axis_index("s") * _NC + lax.axis_index("c")
    ivs = (i0, i1)
    ovs = (o0, o1)
    sis = (si0, si1)
    sos = (so0, so1)
    lane16 = jnp.arange(16, dtype=jnp.int32)

    def stage(i, b):
        uid = wid * _UNITS_W + i
        g = uid // _TB
        tb = uid % _TB
        return pltpu.async_copy(
            idx_hbm.at[g, :, pl.ds(tb * 128, 128)],
            ivs[b].at[pl.ds(0, _U)], sis[b])

    def repack(i, b):
        iv = ivs[b]
        ov = ovs[b]

        def u_body(u, _):
            def bb_body(bb, _):
                v = iv[u, pl.ds(bb * 16, 16)]
                dst = (bb * 16 + lane16) * _U + u
                plsc.store_scatter(ov, [dst], v)
                return 0

            lax.fori_loop(0, 8, bb_body, 0)
            return 0

        lax.fori_loop(0, _U, u_body, 0)

    def flush(i, b):
        uid = wid * _UNITS_W + i
        g = uid // _TB
        tb = uid % _TB
        return pltpu.async_copy(
            ovs[b], out_hbm.at[pl.ds(g * _N + tb * 128 * _U, 128 * _U)],
            sos[b])

    st = [None, None]
    fl = [None, None]
    st[0] = stage(0, 0)
    for i in range(_UNITS_W):
        b = i & 1
        nb = 1 - b
        if i + 1 < _UNITS_W:
            st[nb] = stage(i + 1, nb)
        st[b].wait()
        if fl[b] is not None:
            fl[b].wait()
        repack(i, b)
        fl[b] = flush(i, b)
    for d in fl:
        if d is not None:
            d.wait()


@functools.partial(
    pl.kernel,
    out_type=jax.ShapeDtypeStruct((_B, _U, _D), jnp.float32),
    mesh=_mesh,
    compiler_params=pltpu.CompilerParams(use_tc_tiling_on_sc=False,
                                         needs_layout_passes=False),
    scratch_types=[
        pltpu.VMEM((_P, _C), jnp.int32),
        pltpu.VMEM((_P, _C), jnp.int32),
        pltpu.VMEM((_C, _D), jnp.float32),
        pltpu.VMEM((_C, _D), jnp.float32),
        pltpu.SemaphoreType.DMA,
        pltpu.SemaphoreType.DMA,
        pltpu.SemaphoreType.DMA,
        pltpu.SemaphoreType.DMA,
        pltpu.SemaphoreType.DMA,
        pltpu.SemaphoreType.DMA,
    ],
)
def _gather_sum(idx_hbm, table_hbm, out_hbm, it0, it1, a0, a1,
                si0, si1, sg0, sg1, so0, so1):
    wid = lax.axis_index("s") * _NC + lax.axis_index("c")
    w_row = wid * _ROWS_W
    its = (it0, it1)
    accs = (a0, a1)
    sis = (si0, si1)
    sgs = (sg0, sg1)
    sos = (so0, so1)

    zero16 = jnp.zeros((16,), jnp.float32)

    def stage_idx(k, b):
        base = w_row + k * _C
        return [
            pltpu.async_copy(idx_hbm.at[pl.ds(g * _N + base, _C)],
                             its[b].at[g], sis[b])
            for g in range(_P)
        ]

    def zero_acc(b):
        a = accs[b]

        def z_body(j, _):
            a[j, pl.ds(0, 16)] = zero16
            a[j, pl.ds(16, 16)] = zero16
            return 0

        lax.fori_loop(0, _C, z_body, 0)

    def fire(b):
        return [
            pltpu.async_copy(table_hbm.at[its[b].at[g]], accs[b], sgs[b],
                             add=True)
            for g in range(_P)
        ]

    def fire_out(k, b):
        b0 = wid * _BW + k * _CB
        return [
            pltpu.async_copy(accs[b].at[pl.ds(q * _U, _U)],
                             out_hbm.at[b0 + q], sos[b])
            for q in range(_CB)
        ]

    descs = [None, None]
    out_descs = [None, None]
    st = [None, None]

    st[0] = stage_idx(0, 0)
    zero_acc(0)
    for d in st[0]:
        d.wait()
    descs[0] = fire(0)
    for k in range(_NCHUNK):
        b = k & 1
        nb = 1 - b
        if k + 1 < _NCHUNK:
            st[nb] = stage_idx(k + 1, nb)
            if k + 1 >= 2:
                for d in out_descs[nb]:
                    d.wait()
            zero_acc(nb)
            for d in st[nb]:
                d.wait()
            descs[nb] = fire(nb)
        for d in descs[b]:
            d.wait()
        out_descs[b] = fire_out(k, b)
    for b in ((_NCHUNK - 1) & 1, (_NCHUNK - 2) & 1):
        for d in out_descs[b]:
            d.wait()


def kernel(prev_cmd, num_cmd, ctype_emb_weight):
    del num_cmd
    table = ctype_emb_weight.at[0].set(0.0)
    idx = _detile(prev_cmd.astype(jnp.int32).transpose(2, 1, 0))
    return _gather_sum(idx, table)
